# Initial kernel scaffold; baseline (speedup 1.0000x reference)
import jax
import jax.numpy as jnp
from jax.experimental import pallas as pl


def kernel(in_degree_list, out_degree_list, in_table, out_table):
    raise NotImplementedError("write your pallas kernel here")


# speedup vs baseline: 1.6967x; 1.6967x over previous
import functools

import jax
import jax.numpy as jnp
from jax import lax
from jax.experimental import pallas as pl
from jax.experimental.pallas import tpu as pltpu
from jax.experimental.pallas import tpu_sc as plsc

N_NODES = 50000
HIDDEN = 512
NC = 2
NS = 16
NW = NC * NS
C = 80
NCHUNKS = N_NODES // C
VPR = HIDDEN // 16


def _ce_body(in_idx, out_idx, in_tab, out_tab, out, idx_a, idx_b, buf_a,
             buf_b, sem_a, sem_b):
  wid = lax.axis_index("s") * NC + lax.axis_index("c")

  def chunk_body(c, _):
    base = c * C
    pltpu.sync_copy(in_idx.at[pl.ds(base, C)], idx_a)
    pltpu.sync_copy(out_idx.at[pl.ds(base, C)], idx_b)
    cp_a = pltpu.async_copy(in_tab.at[idx_a], buf_a, sem_a)
    cp_b = pltpu.async_copy(out_tab.at[idx_b], buf_b, sem_b)
    cp_a.wait()
    cp_b.wait()

    def add_row(r, _):
      for k in range(VPR):
        sl = pl.ds(k * 16, 16)
        buf_a[r, sl] = buf_a[r, sl] + buf_b[r, sl]
      return 0

    lax.fori_loop(0, C, add_row, 0)
    pltpu.sync_copy(buf_a, out.at[pl.ds(base, C)])
    return 0

  nw_chunks = (NCHUNKS - wid + NW - 1) // NW
  lax.fori_loop(0, nw_chunks, lambda i, s: chunk_body(wid + i * NW, s), 0)


@jax.jit
def kernel(in_degree_list, out_degree_list, in_table, out_table):
  mesh = plsc.VectorSubcoreMesh(core_axis_name="c", subcore_axis_name="s")
  f = functools.partial(
      pl.kernel,
      out_type=jax.ShapeDtypeStruct((N_NODES, HIDDEN), jnp.float32),
      mesh=mesh,
      scratch_types=[
          pltpu.VMEM((C,), jnp.int32),
          pltpu.VMEM((C,), jnp.int32),
          pltpu.VMEM((C, HIDDEN), jnp.float32),
          pltpu.VMEM((C, HIDDEN), jnp.float32),
          pltpu.SemaphoreType.DMA,
          pltpu.SemaphoreType.DMA,
      ],
  )(_ce_body)
  return f(in_degree_list.astype(jnp.int32), out_degree_list.astype(jnp.int32),
           in_table, out_table)


# speedup vs baseline: 2.4917x; 1.4686x over previous
import functools

import jax
import jax.numpy as jnp
from jax import lax
from jax.experimental import pallas as pl
from jax.experimental.pallas import tpu as pltpu
from jax.experimental.pallas import tpu_sc as plsc

N_NODES = 50000
HIDDEN = 512
NC = 2
NS = 16
NW = NC * NS
C = 40
SZ_BIG = 1600
SZ_SML = 1560
VPR = HIDDEN // 16


def _ce_body(in_idx, out_idx, in_tab, out_tab, out,
             idx_in, idx_out, buf_in0, buf_out0, buf_in1, buf_out1,
             sem_in0, sem_out0, sem_in1, sem_out1):
  wid = lax.axis_index("s") * NC + lax.axis_index("c")
  base = wid * SZ_SML + jnp.minimum(wid, 2) * (SZ_BIG - SZ_SML)
  nw = jnp.where(wid < 2, SZ_BIG // C, SZ_SML // C)

  @pl.when(wid < 2)
  def _():
    pltpu.sync_copy(in_idx.at[pl.ds(base, SZ_BIG)], idx_in)
    pltpu.sync_copy(out_idx.at[pl.ds(base, SZ_BIG)], idx_out)

  @pl.when(wid >= 2)
  def _():
    pltpu.sync_copy(in_idx.at[pl.ds(base, SZ_SML)], idx_in.at[pl.ds(0, SZ_SML)])
    pltpu.sync_copy(out_idx.at[pl.ds(base, SZ_SML)],
                    idx_out.at[pl.ds(0, SZ_SML)])

  bufs = ((buf_in0, buf_out0, sem_in0, sem_out0),
          (buf_in1, buf_out1, sem_in1, sem_out1))

  def issue(j, slot):
    b_in, b_out, s_in, s_out = bufs[slot]

    @pl.when(j < nw)
    def _():
      pltpu.async_copy(in_tab.at[idx_in.at[pl.ds(j * C, C)]], b_in, s_in)
      pltpu.async_copy(out_tab.at[idx_out.at[pl.ds(j * C, C)]], b_out, s_out)

  def process(j, slot):
    b_in, b_out, s_in, s_out = bufs[slot]

    @pl.when(j < nw)
    def _():
      pltpu.make_async_copy(in_tab.at[idx_in.at[pl.ds(j * C, C)]], b_in,
                            s_in).wait()
      pltpu.make_async_copy(out_tab.at[idx_out.at[pl.ds(j * C, C)]], b_out,
                            s_out).wait()

      def add_row(r, _):
        for k in range(VPR):
          sl = pl.ds(k * 16, 16)
          b_in[r, sl] = b_in[r, sl] + b_out[r, sl]
        return 0

      lax.fori_loop(0, C, add_row, 0)
      pltpu.sync_copy(b_in, out.at[pl.ds(base + j * C, C)])

  issue(0, 0)

  def group(g, _):
    for b in range(2):
      j = g * 2 + b
      issue(j + 1, 1 - b)
      process(j, b)
    return 0

  lax.fori_loop(0, (nw + 1) // 2, group, 0)


@jax.jit
def kernel(in_degree_list, out_degree_list, in_table, out_table):
  mesh = plsc.VectorSubcoreMesh(core_axis_name="c", subcore_axis_name="s")
  f = functools.partial(
      pl.kernel,
      out_type=jax.ShapeDtypeStruct((N_NODES, HIDDEN), jnp.float32),
      mesh=mesh,
      scratch_types=[
          pltpu.VMEM((SZ_BIG,), jnp.int32),
          pltpu.VMEM((SZ_BIG,), jnp.int32),
          pltpu.VMEM((C, HIDDEN), jnp.float32),
          pltpu.VMEM((C, HIDDEN), jnp.float32),
          pltpu.VMEM((C, HIDDEN), jnp.float32),
          pltpu.VMEM((C, HIDDEN), jnp.float32),
          pltpu.SemaphoreType.DMA,
          pltpu.SemaphoreType.DMA,
          pltpu.SemaphoreType.DMA,
          pltpu.SemaphoreType.DMA,
      ],
  )(_ce_body)
  return f(in_degree_list.astype(jnp.int32), out_degree_list.astype(jnp.int32),
           in_table, out_table)
